# Initial kernel scaffold; baseline (speedup 1.0000x reference)
import jax
import jax.numpy as jnp
from jax.experimental import pallas as pl


def kernel(flat, cu_seqlens):
    raise NotImplementedError("write your pallas kernel here")


# speedup vs baseline: 1.1814x; 1.1814x over previous
import jax
import jax.numpy as jnp
from jax import lax
from jax.experimental import pallas as pl
from jax.experimental.pallas import tpu as pltpu
from jax.experimental.pallas import tpu_sc as plsc

B = 16
MAX_LEN = 4096
TOTAL = 32768
D = 1024

NW = 32
ROWS_PER_W = (B * MAX_LEN) // NW
HALVES = MAX_LEN // ROWS_PER_W
CHUNK = 64
ZCHUNK = 32
CU_PAD = 32


def _pad_stack_body(flat, cu, zeros, out, cu_v, buf, zbuf):
    c = lax.axis_index("c")
    s = lax.axis_index("s")
    wid = s * 2 + c
    b = wid // HALVES
    h = wid % HALVES
    p0 = h * ROWS_PER_W

    pltpu.sync_copy(cu, cu_v)
    pltpu.sync_copy(zeros, zbuf)

    cu_vec = cu_v[pl.ds(b, 16)]
    cu_b = cu_vec[0]
    cu_b1 = cu_vec[1]
    len_b = cu_b1 - cu_b
    v_total = jnp.clip(len_b - p0, 0, ROWS_PER_W)
    n_full = v_total // CHUNK
    rem = v_total - n_full * CHUNK
    src0 = cu_b + p0

    def copy_chunk(i, carry):
        pltpu.sync_copy(flat.at[pl.ds(src0 + i * CHUNK, CHUNK)], buf)
        pltpu.sync_copy(buf, out.at[b, pl.ds(p0 + i * CHUNK, CHUNK)])
        return carry

    lax.fori_loop(0, n_full, copy_chunk, 0)

    pbase = n_full * CHUNK
    for k in range(5, -1, -1):
        sz = 1 << k
        off = pbase + ((rem >> (k + 1)) << (k + 1))

        @pl.when((rem & sz) != 0)
        def _():
            pltpu.sync_copy(flat.at[pl.ds(src0 + off, sz)],
                            buf.at[pl.ds(0, sz)])
            pltpu.sync_copy(buf.at[pl.ds(0, sz)],
                            out.at[b, pl.ds(p0 + off, sz)])

    z = ROWS_PER_W - v_total
    nz = z // ZCHUNK

    def zero_chunk(i, carry):
        pltpu.sync_copy(zbuf, out.at[b, pl.ds(p0 + v_total + i * ZCHUNK,
                                              ZCHUNK)])
        return carry

    lax.fori_loop(0, nz, zero_chunk, 0)

    zrem = z - nz * ZCHUNK
    zbase = v_total + nz * ZCHUNK
    for k in range(4, -1, -1):
        sz = 1 << k
        off = zbase + ((zrem >> (k + 1)) << (k + 1))

        @pl.when((zrem & sz) != 0)
        def _():
            pltpu.sync_copy(zbuf.at[pl.ds(0, sz)],
                            out.at[b, pl.ds(p0 + off, sz)])


_mesh = plsc.VectorSubcoreMesh(core_axis_name="c", subcore_axis_name="s",
                               num_cores=2, num_subcores=16)

_pad_stack = pl.kernel(
    _pad_stack_body,
    out_type=jax.ShapeDtypeStruct((B, MAX_LEN, D), jnp.float32),
    mesh=_mesh,
    scratch_types=[
        pltpu.VMEM((CU_PAD,), jnp.int32),
        pltpu.VMEM((CHUNK, D), jnp.float32),
        pltpu.VMEM((ZCHUNK, D), jnp.float32),
    ],
    compiler_params=pltpu.CompilerParams(use_tc_tiling_on_sc=False),
)


def kernel(flat, cu_seqlens):
    cu_pad = jnp.zeros((CU_PAD,), jnp.int32).at[: B + 1].set(
        cu_seqlens.astype(jnp.int32))
    zeros = jnp.zeros((ZCHUNK, D), jnp.float32)
    return _pad_stack(flat, cu_pad, zeros)


# speedup vs baseline: 1.2791x; 1.0827x over previous
import jax
import jax.numpy as jnp
from jax import lax
from jax.experimental import pallas as pl
from jax.experimental.pallas import tpu as pltpu
from jax.experimental.pallas import tpu_sc as plsc

B = 16
MAX_LEN = 4096
TOTAL = 32768
D = 1024

NW = 32
ROWS_PER_W = (B * MAX_LEN) // NW
HALVES = MAX_LEN // ROWS_PER_W
CHUNK = 32
NBUF = 3
ZROWS = 256
ZSRC = ZROWS // 16
CU_PAD = 32


def _pad_stack_body(flat, cu, zeros, out, cu_v, buf, zsh, rsem, wsem,
                    zbig_sem, zbit_sem):
    c = lax.axis_index("c")
    s = lax.axis_index("s")
    wid = s * 2 + c
    b = wid // HALVES
    h = wid % HALVES
    p0 = h * ROWS_PER_W

    pltpu.sync_copy(zeros, zsh.at[pl.ds(s * (ZROWS // 16), ZROWS // 16)])
    pltpu.sync_copy(cu, cu_v)
    plsc.subcore_barrier()

    cu_vec = cu_v[pl.ds(b, 16)]
    cu_b = cu_vec[0]
    cu_b1 = cu_vec[1]
    len_b = cu_b1 - cu_b
    v_total = jnp.clip(len_b - p0, 0, ROWS_PER_W)
    n_full = v_total // CHUNK
    rem = v_total - n_full * CHUNK
    src0 = cu_b + p0

    zl = (ZROWS - lax.rem(v_total, ZROWS)) % ZROWS
    zb0 = v_total + zl
    nzb = (ROWS_PER_W - zb0) // ZROWS
    for k in range(7, -1, -1):
        sz = 1 << k
        off = v_total + ((zl >> (k + 1)) << (k + 1))

        @pl.when((zl & sz) != 0)
        def _():
            pltpu.async_copy(zsh.at[pl.ds(0, sz)],
                             out.at[b, pl.ds(p0 + off, sz)],
                             zbit_sem.at[k])
    for j in range(ROWS_PER_W // ZROWS):
        @pl.when(j < nzb)
        def _():
            pltpu.async_copy(zsh,
                             out.at[b, pl.ds(p0 + zb0 + j * ZROWS, ZROWS)],
                             zbig_sem.at[j])

    for pb in range(2):
        @pl.when(pb < n_full)
        def _():
            pltpu.async_copy(flat.at[pl.ds(src0 + pb * CHUNK, CHUNK)],
                             buf.at[pb], rsem.at[pb])

    def copy_step(i, carry):
        slot = lax.rem(i, NBUF)
        pltpu.make_async_copy(flat.at[pl.ds(0, CHUNK)], buf.at[slot],
                              rsem.at[slot]).wait()
        pltpu.async_copy(buf.at[slot],
                         out.at[b, pl.ds(p0 + i * CHUNK, CHUNK)],
                         wsem.at[slot])
        nxt = i + 2
        ws = lax.rem(nxt, NBUF)

        @pl.when(nxt < n_full)
        def _():
            @pl.when(i >= 1)
            def _():
                pltpu.make_async_copy(buf.at[ws],
                                      out.at[b, pl.ds(0, CHUNK)],
                                      wsem.at[ws]).wait()

            pltpu.async_copy(flat.at[pl.ds(src0 + nxt * CHUNK, CHUNK)],
                             buf.at[ws], rsem.at[ws])

        return carry

    lax.fori_loop(0, n_full, copy_step, 0)

    for sl in range(NBUF):
        @pl.when(sl < n_full)
        def _():
            pltpu.make_async_copy(buf.at[sl], out.at[b, pl.ds(0, CHUNK)],
                                  wsem.at[sl]).wait()

    pbase = n_full * CHUNK
    for k in range(4, -1, -1):
        sz = 1 << k
        off = pbase + ((rem >> (k + 1)) << (k + 1))

        @pl.when((rem & sz) != 0)
        def _():
            pltpu.sync_copy(flat.at[pl.ds(src0 + off, sz)],
                            buf.at[0, pl.ds(0, sz)])
            pltpu.sync_copy(buf.at[0, pl.ds(0, sz)],
                            out.at[b, pl.ds(p0 + off, sz)])

    for k in range(7, -1, -1):
        sz = 1 << k

        @pl.when((zl & sz) != 0)
        def _():
            pltpu.make_async_copy(zsh.at[pl.ds(0, sz)],
                                  out.at[b, pl.ds(0, sz)],
                                  zbit_sem.at[k]).wait()
    for j in range(ROWS_PER_W // ZROWS):
        @pl.when(j < nzb)
        def _():
            pltpu.make_async_copy(zsh, out.at[b, pl.ds(0, ZROWS)],
                                  zbig_sem.at[j]).wait()


_mesh = plsc.VectorSubcoreMesh(core_axis_name="c", subcore_axis_name="s",
                               num_cores=2, num_subcores=16)

_pad_stack = pl.kernel(
    _pad_stack_body,
    out_type=jax.ShapeDtypeStruct((B, MAX_LEN, D), jnp.float32),
    mesh=_mesh,
    scratch_types=[
        pltpu.VMEM((CU_PAD,), jnp.int32),
        pltpu.VMEM((NBUF, CHUNK, D), jnp.float32),
        pltpu.VMEM_SHARED((ZROWS, D), jnp.float32),
        pltpu.SemaphoreType.DMA((NBUF,)),
        pltpu.SemaphoreType.DMA((NBUF,)),
        pltpu.SemaphoreType.DMA((ROWS_PER_W // ZROWS,)),
        pltpu.SemaphoreType.DMA((8,)),
    ],
    compiler_params=pltpu.CompilerParams(use_tc_tiling_on_sc=False),
)


def kernel(flat, cu_seqlens):
    cu_pad = jnp.zeros((CU_PAD,), jnp.int32).at[: B + 1].set(
        cu_seqlens.astype(jnp.int32))
    zeros = jnp.zeros((ZSRC, D), jnp.float32)
    return _pad_stack(flat, cu_pad, zeros)
